# Initial kernel scaffold; baseline (speedup 1.0000x reference)
import jax
import jax.numpy as jnp
from jax.experimental import pallas as pl


def kernel(x, node_attrs, edge_index, atomic_numbers, gnb_params):
    raise NotImplementedError("write your pallas kernel here")


# speedup vs baseline: 191.4334x; 191.4334x over previous
import functools
import math

import jax
import jax.numpy as jnp
from jax import lax
from jax.experimental import pallas as pl
from jax.experimental.pallas import tpu as pltpu
from jax.experimental.pallas import tpu_sc as plsc

BJ_A1 = 0.4
BJ_A2 = 4.0

NC = 2
NS = 16
LANES = 128
CH = 1024
K = CH // LANES


def _prep_body(nelem, nz, na_ref, an_ref, gp_ref, elem_ref, tabs_ref):
    m = na_ref[0, :]
    e = jnp.zeros(m.shape, jnp.int32)
    for j in range(1, nelem):
        vj = na_ref[j, :]
        gt = vj > m
        e = jnp.where(gt, j, e)
        m = jnp.where(gt, vj, m)
    elem_ref[0, :] = e

    zz = an_ref[0, :]
    onehot = (zz[:, None] == lax.broadcasted_iota(jnp.int32, (nelem, nz), 1)
              ).astype(jnp.float32)
    p10 = jnp.dot(onehot, gp_ref[...], preferred_element_type=jnp.float32)
    c6 = p10[:, 3]
    rr = p10[:, 2]
    c6ij = jnp.sqrt(c6[:, None] * c6[None, :])
    rij = jnp.sqrt(rr[:, None] * rr[None, :])
    r0 = BJ_A1 * jnp.sqrt(rij) + BJ_A2
    r02 = r0 * r0
    r04 = r02 * r02
    r08 = r04 * r04
    tab_b = 6.0 * (r08 * r04 * r02)
    rij2 = rij * rij
    tab_d = rij2 * rij2 * rij2
    tab_a = -0.5 * c6ij
    tabs_ref[...] = jnp.stack([tab_a, tab_b, tab_d])


def _sc_body(nelem, chunks, per_w, ei_ref, x_ref, elem_ref, tabs_ref, z_ref,
             p_ref, elem_v, ta_v, tb_v, td_v, snd_v, rcv_v, x_v, val_v, vbuf,
             vs):
    c = lax.axis_index("c")
    s = lax.axis_index("s")
    w = s * NC + c
    n_pad = vs.shape[0]
    sz = n_pad // NS

    pltpu.sync_copy(elem_ref, elem_v)
    pltpu.sync_copy(tabs_ref.at[pl.ds(0, LANES)], ta_v)
    pltpu.sync_copy(tabs_ref.at[pl.ds(LANES, LANES)], tb_v)
    pltpu.sync_copy(tabs_ref.at[pl.ds(2 * LANES, LANES)], td_v)

    pltpu.sync_copy(z_ref.at[pl.ds(s * sz, sz)], vbuf)
    pltpu.sync_copy(vbuf, vs.at[pl.ds(s * sz, sz)])

    plsc.subcore_barrier()

    def chunk_body(i, carry):
        cid = w + (NC * NS) * i

        @pl.when(cid < chunks)
        def _do():
            row0 = cid * K
            pltpu.sync_copy(ei_ref.at[0, pl.ds(row0, K)], snd_v)
            pltpu.sync_copy(ei_ref.at[1, pl.ds(row0, K)], rcv_v)
            pltpu.sync_copy(x_ref.at[pl.ds(row0, K)], x_v)
            for k in range(K):
                for j in range(LANES // 16):
                    col = j * 16
                    su = snd_v[k, pl.ds(col, 16)]
                    ru = rcv_v[k, pl.ds(col, 16)]
                    eu = plsc.load_gather(elem_v, [su])
                    ev = plsc.load_gather(elem_v, [ru])
                    pair = eu * nelem + ev
                    a = plsc.load_gather(ta_v, [pair])
                    b = plsc.load_gather(tb_v, [pair])
                    d = plsc.load_gather(td_v, [pair])
                    r = x_v[k, pl.ds(col, 16)]
                    r2 = r * r
                    r4 = r2 * r2
                    r8 = r4 * r4
                    r14 = r8 * r4 * r2
                    r6 = r4 * r2
                    t2 = d + r6
                    den = (r14 + b) * t2
                    q = (a * r14) / den
                    q = jnp.where(t2 > 0.0, q, 0.0)
                    val_v[k, pl.ds(col, 16)] = q
            for k in range(K):
                pltpu.sync_copy(val_v.at[k], vs.at[rcv_v.at[k]], add=True)

        return carry

    lax.fori_loop(0, per_w, chunk_body, 0)
    plsc.subcore_barrier()

    pltpu.sync_copy(vs.at[pl.ds(s * sz, sz)], vbuf)
    pltpu.sync_copy(vbuf, p_ref.at[pl.ds(c * n_pad + s * sz, sz)])


def _add_body(p_ref, v_ref):
    v_ref[0, :] = p_ref[0, :] + p_ref[1, :]


def kernel(x, node_attrs, edge_index, atomic_numbers, gnb_params):
    n = node_attrs.shape[0]
    nelem = node_attrs.shape[1]
    e = x.shape[0]
    nz = gnb_params.shape[0]

    na_t = node_attrs.T
    elem2d, tabs3 = pl.pallas_call(
        functools.partial(_prep_body, nelem, nz),
        out_shape=[
            jax.ShapeDtypeStruct((1, n), jnp.int32),
            jax.ShapeDtypeStruct((3, nelem, nelem), jnp.float32),
        ],
    )(na_t, atomic_numbers.reshape(1, nelem), gnb_params)
    elem = elem2d.reshape(n)
    tabs = jnp.pad(tabs3.reshape(3, nelem * nelem),
                   ((0, 0), (0, LANES - nelem * nelem))).reshape(3 * LANES)

    e_pad = math.ceil(e / CH) * CH
    ei = edge_index
    xf = x.reshape(e)
    if e_pad != e:
        ei = jnp.pad(ei, ((0, 0), (0, e_pad - e)))
        xf = jnp.pad(xf, (0, e_pad - e))
    rows = e_pad // LANES
    chunks = e_pad // CH
    per_w = math.ceil(chunks / (NC * NS))
    ei3 = ei.reshape(2, rows, LANES)
    x2 = xf.reshape(rows, LANES)
    n_pad = math.ceil(n / (NS * 8)) * (NS * 8)
    zeros = jnp.zeros((n_pad,), jnp.float32)

    mesh = plsc.VectorSubcoreMesh(core_axis_name="c", subcore_axis_name="s",
                                  num_cores=NC, num_subcores=NS)
    partials = pl.kernel(
        functools.partial(_sc_body, nelem, chunks, per_w),
        out_type=jax.ShapeDtypeStruct((NC * n_pad,), jnp.float32),
        mesh=mesh,
        compiler_params=pltpu.CompilerParams(needs_layout_passes=False),
        scratch_types=[
            pltpu.VMEM((n,), jnp.int32),
            pltpu.VMEM((LANES,), jnp.float32),
            pltpu.VMEM((LANES,), jnp.float32),
            pltpu.VMEM((LANES,), jnp.float32),
            pltpu.VMEM((K, LANES), jnp.int32),
            pltpu.VMEM((K, LANES), jnp.int32),
            pltpu.VMEM((K, LANES), jnp.float32),
            pltpu.VMEM((K, LANES), jnp.float32),
            pltpu.VMEM((n_pad // NS,), jnp.float32),
            pltpu.VMEM_SHARED((n_pad,), jnp.float32),
        ],
    )(ei3, x2, elem, tabs, zeros)

    v2d = pl.pallas_call(
        _add_body,
        out_shape=jax.ShapeDtypeStruct((1, n_pad), jnp.float32),
    )(partials.reshape(NC, n_pad))
    return v2d.reshape(n_pad)[:n].astype(x.dtype)


# speedup vs baseline: 261.7512x; 1.3673x over previous
import functools
import math

import jax
import jax.numpy as jnp
from jax import lax
from jax.experimental import pallas as pl
from jax.experimental.pallas import tpu as pltpu
from jax.experimental.pallas import tpu_sc as plsc

BJ_A1 = 0.4
BJ_A2 = 4.0

NC = 2
NS = 16
LANES = 128
CH = 1024
K = CH // LANES


def _prep_body(nelem, nz, na_ref, an_ref, gp_ref, elem_ref, tabs_ref):
    m = na_ref[0, :]
    e = jnp.zeros(m.shape, jnp.int32)
    for j in range(1, nelem):
        vj = na_ref[j, :]
        gt = vj > m
        e = jnp.where(gt, j, e)
        m = jnp.where(gt, vj, m)
    elem_ref[0, :] = e

    zz = an_ref[0, :]
    onehot = (zz[:, None] == lax.broadcasted_iota(jnp.int32, (nelem, nz), 1)
              ).astype(jnp.float32)
    p10 = jnp.dot(onehot, gp_ref[...], preferred_element_type=jnp.float32)
    c6 = p10[:, 3]
    rr = p10[:, 2]
    c6ij = jnp.sqrt(c6[:, None] * c6[None, :])
    rij = jnp.sqrt(rr[:, None] * rr[None, :])
    r0 = BJ_A1 * jnp.sqrt(rij) + BJ_A2
    r02 = r0 * r0
    r04 = r02 * r02
    r08 = r04 * r04
    tab_b = 6.0 * (r08 * r04 * r02)
    rij2 = rij * rij
    tab_d = rij2 * rij2 * rij2
    tab_a = -0.5 * c6ij
    tabs_ref[...] = jnp.stack([tab_a, tab_b, tab_d])


def _sc_body(nelem, chunks, per_w, ei_ref, x_ref, elem_ref, tabs_ref, z_ref,
             p_ref, elem_v, ta_v, tb_v, td_v,
             snd0, rcv0, x0, val0, snd1, rcv1, x1, val1, vbuf, vs,
             sem_in0, sem_in1, sem_sc0, sem_sc1):
    c = lax.axis_index("c")
    s = lax.axis_index("s")
    w = s * NC + c
    nw = NC * NS
    n_pad = vs.shape[0]
    sz = n_pad // NS
    bufs = ((snd0, rcv0, x0, val0, sem_in0, sem_sc0),
            (snd1, rcv1, x1, val1, sem_in1, sem_sc1))

    pltpu.sync_copy(elem_ref, elem_v)
    pltpu.sync_copy(tabs_ref.at[pl.ds(0, LANES)], ta_v)
    pltpu.sync_copy(tabs_ref.at[pl.ds(LANES, LANES)], tb_v)
    pltpu.sync_copy(tabs_ref.at[pl.ds(2 * LANES, LANES)], td_v)

    pltpu.sync_copy(z_ref.at[pl.ds(s * sz, sz)], vbuf)
    pltpu.sync_copy(vbuf, vs.at[pl.ds(s * sz, sz)])

    plsc.subcore_barrier()

    def start_inputs(cid, p):
        snd_v, rcv_v, x_v, _, sem_in, _ = bufs[p]
        row0 = cid * K
        pltpu.async_copy(ei_ref.at[0, pl.ds(row0, K)], snd_v, sem_in)
        pltpu.async_copy(ei_ref.at[1, pl.ds(row0, K)], rcv_v, sem_in)
        pltpu.async_copy(x_ref.at[pl.ds(row0, K)], x_v, sem_in)

    def wait_inputs(cid, p):
        snd_v, rcv_v, x_v, _, sem_in, _ = bufs[p]
        row0 = cid * K
        pltpu.make_async_copy(ei_ref.at[0, pl.ds(row0, K)], snd_v, sem_in).wait()
        pltpu.make_async_copy(ei_ref.at[1, pl.ds(row0, K)], rcv_v, sem_in).wait()
        pltpu.make_async_copy(x_ref.at[pl.ds(row0, K)], x_v, sem_in).wait()

    def drain_scatter(p):
        _, rcv_v, _, val_v, _, sem_sc = bufs[p]
        for k in range(K):
            pltpu.make_async_copy(val_v.at[k], vs.at[rcv_v.at[k]], sem_sc).wait()

    def compute_and_fire(p):
        snd_v, rcv_v, x_v, val_v, _, sem_sc = bufs[p]
        for k in range(K):
            for j in range(LANES // 16):
                col = j * 16
                su = snd_v[k, pl.ds(col, 16)]
                ru = rcv_v[k, pl.ds(col, 16)]
                eu = plsc.load_gather(elem_v, [su])
                ev = plsc.load_gather(elem_v, [ru])
                pair = eu * nelem + ev
                a = plsc.load_gather(ta_v, [pair])
                b = plsc.load_gather(tb_v, [pair])
                d = plsc.load_gather(td_v, [pair])
                r = x_v[k, pl.ds(col, 16)]
                r2 = r * r
                r4 = r2 * r2
                r8 = r4 * r4
                r14 = r8 * r4 * r2
                r6 = r4 * r2
                t2 = d + r6
                den = (r14 + b) * t2
                q = (a * r14) / den
                q = jnp.where(t2 > 0.0, q, 0.0)
                val_v[k, pl.ds(col, 16)] = q
        for k in range(K):
            pltpu.async_copy(val_v.at[k], vs.at[rcv_v.at[k]], sem_sc, add=True)

    def stage(i2, idx_off, p):
        idx = 2 * i2 + idx_off
        cid = w + nw * idx

        @pl.when(cid < chunks)
        def _do():
            wait_inputs(cid, p)

            @pl.when(idx >= 1)
            def _drain_prev():
                drain_scatter(1 - p)

            @pl.when(cid + nw < chunks)
            def _prefetch():
                start_inputs(cid + nw, 1 - p)

            compute_and_fire(p)

    @pl.when(w < chunks)
    def _prime():
        start_inputs(w, 0)

    def chunk_body(i2, carry):
        stage(i2, 0, 0)
        stage(i2, 1, 1)
        return carry

    lax.fori_loop(0, (per_w + 1) // 2, chunk_body, 0)

    nch = (chunks - w + nw - 1) // nw
    last_parity = (nch - 1) % 2

    @pl.when(last_parity == 0)
    def _d0():
        drain_scatter(0)

    @pl.when(last_parity == 1)
    def _d1():
        drain_scatter(1)

    plsc.subcore_barrier()

    pltpu.sync_copy(vs.at[pl.ds(s * sz, sz)], vbuf)
    pltpu.sync_copy(vbuf, p_ref.at[pl.ds(c * n_pad + s * sz, sz)])


def _add_body(p_ref, v_ref):
    v_ref[0, :] = p_ref[0, :] + p_ref[1, :]


def kernel(x, node_attrs, edge_index, atomic_numbers, gnb_params):
    n = node_attrs.shape[0]
    nelem = node_attrs.shape[1]
    e = x.shape[0]
    nz = gnb_params.shape[0]

    na_t = node_attrs.T
    elem2d, tabs3 = pl.pallas_call(
        functools.partial(_prep_body, nelem, nz),
        out_shape=[
            jax.ShapeDtypeStruct((1, n), jnp.int32),
            jax.ShapeDtypeStruct((3, nelem, nelem), jnp.float32),
        ],
    )(na_t, atomic_numbers.reshape(1, nelem), gnb_params)
    elem = elem2d.reshape(n)
    tabs = jnp.pad(tabs3.reshape(3, nelem * nelem),
                   ((0, 0), (0, LANES - nelem * nelem))).reshape(3 * LANES)

    e_pad = math.ceil(e / CH) * CH
    ei = edge_index
    xf = x.reshape(e)
    if e_pad != e:
        ei = jnp.pad(ei, ((0, 0), (0, e_pad - e)))
        xf = jnp.pad(xf, (0, e_pad - e))
    rows = e_pad // LANES
    chunks = e_pad // CH
    per_w = math.ceil(chunks / (NC * NS))
    ei3 = ei.reshape(2, rows, LANES)
    x2 = xf.reshape(rows, LANES)
    n_pad = math.ceil(n / (NS * 8)) * (NS * 8)
    zeros = jnp.zeros((n_pad,), jnp.float32)

    mesh = plsc.VectorSubcoreMesh(core_axis_name="c", subcore_axis_name="s",
                                  num_cores=NC, num_subcores=NS)
    partials = pl.kernel(
        functools.partial(_sc_body, nelem, chunks, per_w),
        out_type=jax.ShapeDtypeStruct((NC * n_pad,), jnp.float32),
        mesh=mesh,
        compiler_params=pltpu.CompilerParams(needs_layout_passes=False),
        scratch_types=(
            [pltpu.VMEM((n,), jnp.int32)]
            + [pltpu.VMEM((LANES,), jnp.float32)] * 3
            + [pltpu.VMEM((K, LANES), jnp.int32),
               pltpu.VMEM((K, LANES), jnp.int32),
               pltpu.VMEM((K, LANES), jnp.float32),
               pltpu.VMEM((K, LANES), jnp.float32)] * 2
            + [pltpu.VMEM((n_pad // NS,), jnp.float32),
               pltpu.VMEM_SHARED((n_pad,), jnp.float32)]
            + [pltpu.SemaphoreType.DMA] * 4
        ),
    )(ei3, x2, elem, tabs, zeros)

    v2d = pl.pallas_call(
        _add_body,
        out_shape=jax.ShapeDtypeStruct((1, n_pad), jnp.float32),
    )(partials.reshape(NC, n_pad))
    return v2d.reshape(n_pad)[:n].astype(x.dtype)
